# baseline (speedup 1.0000x reference)
import functools

import jax
import jax.numpy as jnp
from jax import lax
from jax.experimental import pallas as pl
from jax.experimental.pallas import tpu as pltpu
from jax.experimental.pallas import tpu_sc as plsc

B = 8
N = 4096
S = 512
K = 32
R2 = 0.2 ** 2
F_IN = 128
F_MID = 256
F_OUT = 512


def _fps_body(initf_ref, x_ref, y_ref, z_ref, idx_ref, cx_ref, cy_ref, cz_ref):
    x = x_ref[...]
    y = y_ref[...]
    z = z_ref[...]
    iota = lax.broadcasted_iota(jnp.int32, (B, N), 1)
    iota_s = lax.broadcasted_iota(jnp.int32, (B, S), 1)

    def step(i, carry):
        distance, far, aidx, acx, acy, acz = carry
        onehot = iota == far
        cx = jnp.sum(jnp.where(onehot, x, 0.0), axis=1, keepdims=True)
        cy = jnp.sum(jnp.where(onehot, y, 0.0), axis=1, keepdims=True)
        cz = jnp.sum(jnp.where(onehot, z, 0.0), axis=1, keepdims=True)
        hit = iota_s == i
        aidx = jnp.where(hit, far, aidx)
        acx = jnp.where(hit, cx, acx)
        acy = jnp.where(hit, cy, acy)
        acz = jnp.where(hit, cz, acz)
        dx = x - cx
        dy = y - cy
        dz = z - cz
        d = (dx * dx + dy * dy) + dz * dz
        distance = jnp.minimum(distance, d)
        m = jnp.max(distance, axis=1, keepdims=True)
        far_new = jnp.min(jnp.where(distance == m, iota, N), axis=1,
                          keepdims=True)
        return distance, far_new, aidx, acx, acy, acz

    dist0 = jnp.full((B, N), 1e10, dtype=jnp.float32)
    far0 = initf_ref[:, 0:1]
    zi = jnp.zeros((B, S), jnp.int32)
    zf = jnp.zeros((B, S), jnp.float32)
    _, _, aidx, acx, acy, acz = lax.fori_loop(
        0, S, step, (dist0, far0, zi, zf, zf, zf))
    idx_ref[...] = aidx
    cx_ref[...] = acx
    cy_ref[...] = acy
    cz_ref[...] = acz


def _run_fps(xT, init_far):
    out_shapes = (
        jax.ShapeDtypeStruct((B, S), jnp.int32),
        jax.ShapeDtypeStruct((B, S), jnp.float32),
        jax.ShapeDtypeStruct((B, S), jnp.float32),
        jax.ShapeDtypeStruct((B, S), jnp.float32),
    )
    return pl.pallas_call(
        _fps_body,
        out_shape=out_shapes,
    )(init_far, xT[:, 0], xT[:, 1], xT[:, 2])


def _px_body(pts_ref, a8_ref, w1p_ref, w1x_ref, b1_ref, out_ref):
    acc = jnp.dot(pts_ref[...], w1p_ref[...],
                  preferred_element_type=jnp.float32)
    acc = acc + jnp.dot(a8_ref[...], w1x_ref[...],
                        preferred_element_type=jnp.float32)
    out_ref[...] = acc + b1_ref[...]


def _run_px(points_flat, xyz8, W1p, W1x8, b1):
    bm = 1024
    grid = (B * N // bm,)
    return pl.pallas_call(
        _px_body,
        grid=grid,
        compiler_params=pltpu.CompilerParams(
            dimension_semantics=("parallel",)),
        in_specs=[
            pl.BlockSpec((bm, F_IN), lambda i: (i, 0)),
            pl.BlockSpec((bm, 8), lambda i: (i, 0)),
            pl.BlockSpec((F_IN, F_MID), lambda i: (0, 0)),
            pl.BlockSpec((8, F_MID), lambda i: (0, 0)),
            pl.BlockSpec((1, F_MID), lambda i: (0, 0)),
        ],
        out_specs=pl.BlockSpec((bm, F_MID), lambda i: (i, 0)),
        out_shape=jax.ShapeDtypeStruct((B * N, F_MID), jnp.float32),
    )(points_flat, xyz8, W1p, W1x8, b1)


def _bq_body(x_ref, y_ref, z_ref, xb8_ref, cx_ref, cy_ref, cz_ref, out_ref,
             s_scr):
    b = pl.program_id(0)
    x = x_ref[0]
    y = y_ref[0]
    z = z_ref[0]
    xb8 = xb8_ref[0]
    pn = (x * x + y * y) + z * z
    cb = jnp.concatenate(
        [cx_ref[0], cy_ref[0], cz_ref[0], jnp.zeros((S, 5), jnp.float32)],
        axis=1).astype(jnp.bfloat16)
    s_scr[...] = jnp.dot(cb, xb8, preferred_element_type=jnp.float32)

    def chunk(sc, _):
        r = pl.ds(pl.multiple_of(sc * 8, 8), 8)
        cx = cx_ref[0, r, :]
        cy = cy_ref[0, r, :]
        cz = cz_ref[0, r, :]
        cn = (cx * cx + cy * cy) + cz * cz
        s = s_scr[r, :]
        d = (-2.0 * s + cn) + pn
        m = (d <= R2).astype(jnp.int32)
        cs = m
        sh = 1
        while sh < N:
            z128 = jnp.zeros((8, sh), jnp.int32)
            cs = cs + jnp.concatenate([z128, cs[:, :-sh]], axis=1)
            sh *= 2
        cols = []
        for k in range(K):
            cols.append(jnp.sum((cs <= k).astype(jnp.int32), axis=1))
        cnt = jnp.stack(cols, axis=1)
        first = cnt[:, 0:1]
        sel = jnp.where(cnt == N, first, cnt) + b * N
        out_ref[0, r, :] = sel
        return 0

    lax.fori_loop(0, S // 8, chunk, 0)


def _run_bq(xT, xb8, cx, cy, cz):
    return pl.pallas_call(
        _bq_body,
        grid=(B,),
        compiler_params=pltpu.CompilerParams(
            dimension_semantics=("parallel",)),
        in_specs=[
            pl.BlockSpec((1, 1, N), lambda b: (b, 0, 0)),
            pl.BlockSpec((1, 1, N), lambda b: (b, 0, 0)),
            pl.BlockSpec((1, 1, N), lambda b: (b, 0, 0)),
            pl.BlockSpec((1, 8, N), lambda b: (b, 0, 0)),
            pl.BlockSpec((1, S, 1), lambda b: (b, 0, 0)),
            pl.BlockSpec((1, S, 1), lambda b: (b, 0, 0)),
            pl.BlockSpec((1, S, 1), lambda b: (b, 0, 0)),
        ],
        out_specs=pl.BlockSpec((1, S, K), lambda b: (b, 0, 0)),
        out_shape=jax.ShapeDtypeStruct((B, S, K), jnp.int32),
        scratch_shapes=[pltpu.VMEM((S, N), jnp.float32)],
    )(xT[:, 0:1], xT[:, 1:2], xT[:, 2:3], xb8,
      cx.reshape(B, S, 1), cy.reshape(B, S, 1), cz.reshape(B, S, 1))


def _run_sc_gather(px_flat, gidx_flat, pts_flat, fps_flat):
    NW = 32
    G_ROWS = B * S * K
    CH = 128
    g_per_w = G_ROWS // NW
    id_per_w = B * S // NW
    mesh = plsc.VectorSubcoreMesh(core_axis_name="c", subcore_axis_name="s")

    @functools.partial(
        pl.kernel,
        mesh=mesh,
        out_type=[
            jax.ShapeDtypeStruct((G_ROWS, F_MID), jnp.float32),
            jax.ShapeDtypeStruct((B * S, F_IN), jnp.float32),
        ],
        scratch_types=[
            pltpu.VMEM((CH,), jnp.int32),
            pltpu.VMEM((CH, F_MID), jnp.float32),
            pltpu.VMEM((id_per_w,), jnp.int32),
            pltpu.VMEM((id_per_w, F_IN), jnp.float32),
            pltpu.SemaphoreType.DMA,
        ],
    )
    def sc_gather(px_hbm, gidx_hbm, pts_hbm, fps_hbm, g_out, id_out,
                  idx_v, rows_v, idx2_v, rows2_v, sem):
        wid = lax.axis_index("s") * 2 + lax.axis_index("c")

        base2 = wid * id_per_w
        pltpu.sync_copy(fps_hbm.at[pl.ds(base2, id_per_w)], idx2_v)
        pltpu.async_copy(pts_hbm.at[idx2_v], rows2_v, sem).wait()
        pltpu.sync_copy(rows2_v, id_out.at[pl.ds(base2, id_per_w)])

        def body(c, _):
            base = wid * g_per_w + c * CH
            pltpu.sync_copy(gidx_hbm.at[pl.ds(base, CH)], idx_v)
            pltpu.async_copy(px_hbm.at[idx_v], rows_v, sem).wait()
            pltpu.sync_copy(rows_v, g_out.at[pl.ds(base, CH)])
            return 0

        lax.fori_loop(0, g_per_w // CH, body, 0)

    return sc_gather(px_flat, gidx_flat, pts_flat, fps_flat)


def _mlp_body(g_ref, n8_ref, id_ref, w1x_ref, w2_ref, wsk_ref, b2_ref,
              bsk_ref, out_ref):
    SB = n8_ref.shape[1]
    c1 = jnp.dot(n8_ref[0], w1x_ref[...],
                 preferred_element_type=jnp.float32)
    g = g_ref[0].reshape(SB, K, F_MID)
    h1 = jnp.maximum(g - c1[:, None, :], 0.0).reshape(SB * K, F_MID)
    h2 = jnp.dot(h1.astype(jnp.bfloat16), w2_ref[...].astype(jnp.bfloat16),
                 preferred_element_type=jnp.float32)
    m = jnp.max(h2.reshape(SB, K, F_OUT), axis=1)
    skip = jnp.dot(id_ref[0].astype(jnp.bfloat16),
                   wsk_ref[...].astype(jnp.bfloat16),
                   preferred_element_type=jnp.float32)
    out_ref[0] = jnp.maximum(m + b2_ref[...] + skip + bsk_ref[...], 0.0)


def _run_mlp(g, nxyz8, identity, W1x8, W2, Wskip, b2, bskip):
    SB = 32
    grid = (B, S // SB)
    return pl.pallas_call(
        _mlp_body,
        grid=grid,
        compiler_params=pltpu.CompilerParams(
            dimension_semantics=("parallel", "parallel")),
        in_specs=[
            pl.BlockSpec((1, SB * K, F_MID), lambda b, s: (b, s, 0)),
            pl.BlockSpec((1, SB, 8), lambda b, s: (b, s, 0)),
            pl.BlockSpec((1, SB, F_IN), lambda b, s: (b, s, 0)),
            pl.BlockSpec((8, F_MID), lambda b, s: (0, 0)),
            pl.BlockSpec((F_MID, F_OUT), lambda b, s: (0, 0)),
            pl.BlockSpec((F_IN, F_OUT), lambda b, s: (0, 0)),
            pl.BlockSpec((1, F_OUT), lambda b, s: (0, 0)),
            pl.BlockSpec((1, F_OUT), lambda b, s: (0, 0)),
        ],
        out_specs=pl.BlockSpec((1, SB, F_OUT), lambda b, s: (b, s, 0)),
        out_shape=jax.ShapeDtypeStruct((B, S, F_OUT), jnp.float32),
    )(g, nxyz8, identity, W1x8, W2, Wskip, b2, bskip)


def kernel(xyz, points, W1, b1, W2, b2, Wskip, bskip):
    fps_key = jax.random.key(42)
    init_far = jax.random.randint(fps_key, (B,), 0, N, dtype=jnp.int32)
    init_far = jnp.broadcast_to(init_far[:, None], (B, 128))

    xT = jnp.transpose(xyz, (0, 2, 1))
    fps_idx, cx, cy, cz = _run_fps(xT, init_far)
    new_xyz = jnp.stack([cx, cy, cz], axis=-1)

    W1x = W1[:3]
    W1p = W1[3:]
    W1x8 = jnp.pad(W1x, ((0, 5), (0, 0)))
    xyz8 = jnp.pad(xyz.reshape(B * N, 3), ((0, 0), (0, 5)))
    px = _run_px(points.reshape(B * N, F_IN), xyz8, W1p, W1x8,
                 b1.reshape(1, F_MID))

    xb8 = jnp.concatenate(
        [xT.astype(jnp.bfloat16),
         jnp.zeros((B, 5, N), jnp.bfloat16)], axis=1)
    gidx = _run_bq(xT, xb8, cx, cy, cz)
    fps_flat = (fps_idx + jnp.arange(B, dtype=jnp.int32)[:, None] * N)

    g_flat, id_flat = _run_sc_gather(
        px, gidx.reshape(B * S * K), points.reshape(B * N, F_IN),
        fps_flat.reshape(B * S))

    nxyz8 = jnp.pad(new_xyz, ((0, 0), (0, 0), (0, 5)))
    x = _run_mlp(g_flat.reshape(B, S * K, F_MID), nxyz8,
                 id_flat.reshape(B, S, F_IN), W1x8, W2, Wskip,
                 b2.reshape(1, F_OUT), bskip.reshape(1, F_OUT))
    return (new_xyz, x)


# speedup vs baseline: 1.0531x; 1.0531x over previous
import functools

import jax
import jax.numpy as jnp
from jax import lax
from jax.experimental import pallas as pl
from jax.experimental.pallas import tpu as pltpu
from jax.experimental.pallas import tpu_sc as plsc

B = 8
N = 4096
S = 512
K = 32
R2 = 0.2 ** 2
F_IN = 128
F_MID = 256
F_OUT = 512


def _fps_body(initf_ref, x_ref, y_ref, z_ref, idx_ref, cx_ref, cy_ref, cz_ref):
    x = x_ref[...]
    y = y_ref[...]
    z = z_ref[...]
    iota = lax.broadcasted_iota(jnp.int32, (B, N), 1)
    iota_s = lax.broadcasted_iota(jnp.int32, (B, S), 1)

    def step(i, carry):
        distance, far, aidx, acx, acy, acz = carry
        onehot = iota == far
        cx = jnp.sum(jnp.where(onehot, x, 0.0), axis=1, keepdims=True)
        cy = jnp.sum(jnp.where(onehot, y, 0.0), axis=1, keepdims=True)
        cz = jnp.sum(jnp.where(onehot, z, 0.0), axis=1, keepdims=True)
        hit = iota_s == i
        aidx = jnp.where(hit, far, aidx)
        acx = jnp.where(hit, cx, acx)
        acy = jnp.where(hit, cy, acy)
        acz = jnp.where(hit, cz, acz)
        dx = x - cx
        dy = y - cy
        dz = z - cz
        d = (dx * dx + dy * dy) + dz * dz
        distance = jnp.minimum(distance, d)
        m = jnp.max(distance, axis=1, keepdims=True)
        far_new = jnp.min(jnp.where(distance == m, iota, N), axis=1,
                          keepdims=True)
        return distance, far_new, aidx, acx, acy, acz

    dist0 = jnp.full((B, N), 1e10, dtype=jnp.float32)
    far0 = initf_ref[:, 0:1]
    zi = jnp.zeros((B, S), jnp.int32)
    zf = jnp.zeros((B, S), jnp.float32)
    _, _, aidx, acx, acy, acz = lax.fori_loop(
        0, S, step, (dist0, far0, zi, zf, zf, zf))
    idx_ref[...] = aidx
    cx_ref[...] = acx
    cy_ref[...] = acy
    cz_ref[...] = acz


def _run_fps(xT, init_far):
    out_shapes = (
        jax.ShapeDtypeStruct((B, S), jnp.int32),
        jax.ShapeDtypeStruct((B, S), jnp.float32),
        jax.ShapeDtypeStruct((B, S), jnp.float32),
        jax.ShapeDtypeStruct((B, S), jnp.float32),
    )
    return pl.pallas_call(
        _fps_body,
        out_shape=out_shapes,
    )(init_far, xT[:, 0], xT[:, 1], xT[:, 2])


def _px_body(pts_ref, a8_ref, w1p_ref, w1x_ref, b1_ref, out_ref):
    acc = jnp.dot(pts_ref[...], w1p_ref[...],
                  preferred_element_type=jnp.float32)
    acc = acc + jnp.dot(a8_ref[...], w1x_ref[...],
                        preferred_element_type=jnp.float32)
    bq = (acc + b1_ref[...]).astype(jnp.bfloat16).astype(jnp.float32)
    u = lax.bitcast_convert_type(bq, jnp.uint32)
    lo = u[:, :128] >> 16
    hi = (u[:, 128:] >> 16) << 16
    out_ref[...] = hi | lo


def _run_px(points_flat, xyz8, W1p, W1x8, b1):
    bm = 1024
    grid = (B * N // bm,)
    return pl.pallas_call(
        _px_body,
        grid=grid,
        compiler_params=pltpu.CompilerParams(
            dimension_semantics=("parallel",)),
        in_specs=[
            pl.BlockSpec((bm, F_IN), lambda i: (i, 0)),
            pl.BlockSpec((bm, 8), lambda i: (i, 0)),
            pl.BlockSpec((F_IN, F_MID), lambda i: (0, 0)),
            pl.BlockSpec((8, F_MID), lambda i: (0, 0)),
            pl.BlockSpec((1, F_MID), lambda i: (0, 0)),
        ],
        out_specs=pl.BlockSpec((bm, F_MID // 2), lambda i: (i, 0)),
        out_shape=jax.ShapeDtypeStruct((B * N, F_MID // 2), jnp.uint32),
    )(points_flat, xyz8, W1p, W1x8, b1)


def _bq_body(x_ref, y_ref, z_ref, xb8_ref, cx_ref, cy_ref, cz_ref, out_ref,
             s_scr):
    b = pl.program_id(0)
    x = x_ref[0]
    y = y_ref[0]
    z = z_ref[0]
    xb8 = xb8_ref[0]
    pn = (x * x + y * y) + z * z
    cb = jnp.concatenate(
        [cx_ref[0], cy_ref[0], cz_ref[0], jnp.zeros((S, 5), jnp.float32)],
        axis=1).astype(jnp.bfloat16)
    s_scr[...] = jnp.dot(cb, xb8, preferred_element_type=jnp.float32)

    def chunk(sc, _):
        r = pl.ds(pl.multiple_of(sc * 8, 8), 8)
        cx = cx_ref[0, r, :]
        cy = cy_ref[0, r, :]
        cz = cz_ref[0, r, :]
        cn = (cx * cx + cy * cy) + cz * cz
        s = s_scr[r, :]
        d = (-2.0 * s + cn) + pn
        m = (d <= R2).astype(jnp.int32)
        cs = m
        sh = 1
        while sh < N:
            z128 = jnp.zeros((8, sh), jnp.int32)
            cs = cs + jnp.concatenate([z128, cs[:, :-sh]], axis=1)
            sh *= 2
        cols = []
        for k in range(K):
            cols.append(jnp.sum((cs <= k).astype(jnp.int32), axis=1))
        cnt = jnp.stack(cols, axis=1)
        first = cnt[:, 0:1]
        sel = jnp.where(cnt == N, first, cnt) + b * N
        out_ref[0, r, :] = sel
        return 0

    lax.fori_loop(0, S // 8, chunk, 0)


def _run_bq(xT, xb8, cx, cy, cz):
    return pl.pallas_call(
        _bq_body,
        grid=(B,),
        compiler_params=pltpu.CompilerParams(
            dimension_semantics=("parallel",)),
        in_specs=[
            pl.BlockSpec((1, 1, N), lambda b: (b, 0, 0)),
            pl.BlockSpec((1, 1, N), lambda b: (b, 0, 0)),
            pl.BlockSpec((1, 1, N), lambda b: (b, 0, 0)),
            pl.BlockSpec((1, 8, N), lambda b: (b, 0, 0)),
            pl.BlockSpec((1, S, 1), lambda b: (b, 0, 0)),
            pl.BlockSpec((1, S, 1), lambda b: (b, 0, 0)),
            pl.BlockSpec((1, S, 1), lambda b: (b, 0, 0)),
        ],
        out_specs=pl.BlockSpec((1, S, K), lambda b: (b, 0, 0)),
        out_shape=jax.ShapeDtypeStruct((B, S, K), jnp.int32),
        scratch_shapes=[pltpu.VMEM((S, N), jnp.float32)],
    )(xT[:, 0:1], xT[:, 1:2], xT[:, 2:3], xb8,
      cx.reshape(B, S, 1), cy.reshape(B, S, 1), cz.reshape(B, S, 1))


def _run_sc_gather(px_flat, gidx_flat, pts_flat, fps_flat):
    NW = 32
    G_ROWS = B * S * K
    CH = 128
    g_per_w = G_ROWS // NW
    id_per_w = B * S // NW
    mesh = plsc.VectorSubcoreMesh(core_axis_name="c", subcore_axis_name="s")

    @functools.partial(
        pl.kernel,
        mesh=mesh,
        out_type=[
            jax.ShapeDtypeStruct((G_ROWS, F_MID // 2), jnp.uint32),
            jax.ShapeDtypeStruct((B * S, F_IN), jnp.float32),
        ],
        scratch_types=[
            pltpu.VMEM((CH,), jnp.int32),
            pltpu.VMEM((CH, F_MID // 2), jnp.uint32),
            pltpu.VMEM((id_per_w,), jnp.int32),
            pltpu.VMEM((id_per_w, F_IN), jnp.float32),
            pltpu.SemaphoreType.DMA,
        ],
    )
    def sc_gather(px_hbm, gidx_hbm, pts_hbm, fps_hbm, g_out, id_out,
                  idx_v, rows_v, idx2_v, rows2_v, sem):
        wid = lax.axis_index("s") * 2 + lax.axis_index("c")

        base2 = wid * id_per_w
        pltpu.sync_copy(fps_hbm.at[pl.ds(base2, id_per_w)], idx2_v)
        pltpu.async_copy(pts_hbm.at[idx2_v], rows2_v, sem).wait()
        pltpu.sync_copy(rows2_v, id_out.at[pl.ds(base2, id_per_w)])

        def body(c, _):
            base = wid * g_per_w + c * CH
            pltpu.sync_copy(gidx_hbm.at[pl.ds(base, CH)], idx_v)
            pltpu.async_copy(px_hbm.at[idx_v], rows_v, sem).wait()
            pltpu.sync_copy(rows_v, g_out.at[pl.ds(base, CH)])
            return 0

        lax.fori_loop(0, g_per_w // CH, body, 0)

    return sc_gather(px_flat, gidx_flat, pts_flat, fps_flat)


def _mlp_body(g_ref, n8_ref, id_ref, w1x_ref, w2_ref, wsk_ref, b2_ref,
              bsk_ref, out_ref):
    SB = n8_ref.shape[1]
    c1 = jnp.dot(n8_ref[0], w1x_ref[...],
                 preferred_element_type=jnp.float32)
    gu = g_ref[0]
    f_lo = lax.bitcast_convert_type(gu << 16, jnp.float32)
    f_hi = lax.bitcast_convert_type((gu >> 16) << 16, jnp.float32)
    g = jnp.concatenate([f_lo, f_hi], axis=1).reshape(SB, K, F_MID)
    h1 = jnp.maximum(g - c1[:, None, :], 0.0).reshape(SB * K, F_MID)
    h2 = jnp.dot(h1.astype(jnp.bfloat16), w2_ref[...].astype(jnp.bfloat16),
                 preferred_element_type=jnp.float32)
    m = jnp.max(h2.reshape(SB, K, F_OUT), axis=1)
    skip = jnp.dot(id_ref[0].astype(jnp.bfloat16),
                   wsk_ref[...].astype(jnp.bfloat16),
                   preferred_element_type=jnp.float32)
    out_ref[0] = jnp.maximum(m + b2_ref[...] + skip + bsk_ref[...], 0.0)


def _run_mlp(g, nxyz8, identity, W1x8, W2, Wskip, b2, bskip):
    SB = 32
    grid = (B, S // SB)
    return pl.pallas_call(
        _mlp_body,
        grid=grid,
        compiler_params=pltpu.CompilerParams(
            dimension_semantics=("parallel", "parallel")),
        in_specs=[
            pl.BlockSpec((1, SB * K, F_MID // 2), lambda b, s: (b, s, 0)),
            pl.BlockSpec((1, SB, 8), lambda b, s: (b, s, 0)),
            pl.BlockSpec((1, SB, F_IN), lambda b, s: (b, s, 0)),
            pl.BlockSpec((8, F_MID), lambda b, s: (0, 0)),
            pl.BlockSpec((F_MID, F_OUT), lambda b, s: (0, 0)),
            pl.BlockSpec((F_IN, F_OUT), lambda b, s: (0, 0)),
            pl.BlockSpec((1, F_OUT), lambda b, s: (0, 0)),
            pl.BlockSpec((1, F_OUT), lambda b, s: (0, 0)),
        ],
        out_specs=pl.BlockSpec((1, SB, F_OUT), lambda b, s: (b, s, 0)),
        out_shape=jax.ShapeDtypeStruct((B, S, F_OUT), jnp.float32),
    )(g, nxyz8, identity, W1x8, W2, Wskip, b2, bskip)


def kernel(xyz, points, W1, b1, W2, b2, Wskip, bskip):
    fps_key = jax.random.key(42)
    init_far = jax.random.randint(fps_key, (B,), 0, N, dtype=jnp.int32)
    init_far = jnp.broadcast_to(init_far[:, None], (B, 128))

    xT = jnp.transpose(xyz, (0, 2, 1))
    fps_idx, cx, cy, cz = _run_fps(xT, init_far)
    new_xyz = jnp.stack([cx, cy, cz], axis=-1)

    W1x = W1[:3]
    W1p = W1[3:]
    W1x8 = jnp.pad(W1x, ((0, 5), (0, 0)))
    xyz8 = jnp.pad(xyz.reshape(B * N, 3), ((0, 0), (0, 5)))
    px = _run_px(points.reshape(B * N, F_IN), xyz8, W1p, W1x8,
                 b1.reshape(1, F_MID))

    xb8 = jnp.concatenate(
        [xT.astype(jnp.bfloat16),
         jnp.zeros((B, 5, N), jnp.bfloat16)], axis=1)
    gidx = _run_bq(xT, xb8, cx, cy, cz)
    fps_flat = (fps_idx + jnp.arange(B, dtype=jnp.int32)[:, None] * N)

    g_flat, id_flat = _run_sc_gather(
        px, gidx.reshape(B * S * K), points.reshape(B * N, F_IN),
        fps_flat.reshape(B * S))

    nxyz8 = jnp.pad(new_xyz, ((0, 0), (0, 0), (0, 5)))
    x = _run_mlp(g_flat.reshape(B, S * K, F_MID // 2), nxyz8,
                 id_flat.reshape(B, S, F_IN), W1x8, W2, Wskip,
                 b2.reshape(1, F_OUT), bskip.reshape(1, F_OUT))
    return (new_xyz, x)


# speedup vs baseline: 1.2863x; 1.2215x over previous
import functools

import jax
import jax.numpy as jnp
from jax import lax
from jax.experimental import pallas as pl
from jax.experimental.pallas import tpu as pltpu
from jax.experimental.pallas import tpu_sc as plsc

B = 8
N = 4096
S = 512
K = 32
R2 = 0.2 ** 2
F_IN = 128
F_MID = 256
F_OUT = 512


def _fps_body(initf_ref, x_ref, y_ref, z_ref, idx_ref, cx_ref, cy_ref, cz_ref):
    x = x_ref[...]
    y = y_ref[...]
    z = z_ref[...]
    iota = lax.broadcasted_iota(jnp.int32, (B, N), 1)
    iota_s = lax.broadcasted_iota(jnp.int32, (B, S), 1)

    def step(i, carry):
        distance, far, aidx, acx, acy, acz = carry
        onehot = iota == far
        cx = jnp.sum(jnp.where(onehot, x, 0.0), axis=1, keepdims=True)
        cy = jnp.sum(jnp.where(onehot, y, 0.0), axis=1, keepdims=True)
        cz = jnp.sum(jnp.where(onehot, z, 0.0), axis=1, keepdims=True)
        hit = iota_s == i
        aidx = jnp.where(hit, far, aidx)
        acx = jnp.where(hit, cx, acx)
        acy = jnp.where(hit, cy, acy)
        acz = jnp.where(hit, cz, acz)
        dx = x - cx
        dy = y - cy
        dz = z - cz
        d = (dx * dx + dy * dy) + dz * dz
        distance = jnp.minimum(distance, d)
        m = jnp.max(distance, axis=1, keepdims=True)
        far_new = jnp.min(jnp.where(distance == m, iota, N), axis=1,
                          keepdims=True)
        return distance, far_new, aidx, acx, acy, acz

    dist0 = jnp.full((B, N), 1e10, dtype=jnp.float32)
    far0 = initf_ref[:, 0:1]
    zi = jnp.zeros((B, S), jnp.int32)
    zf = jnp.zeros((B, S), jnp.float32)
    _, _, aidx, acx, acy, acz = lax.fori_loop(
        0, S, step, (dist0, far0, zi, zf, zf, zf))
    idx_ref[...] = aidx
    cx_ref[...] = acx
    cy_ref[...] = acy
    cz_ref[...] = acz


def _run_fps(xT, init_far):
    out_shapes = (
        jax.ShapeDtypeStruct((B, S), jnp.int32),
        jax.ShapeDtypeStruct((B, S), jnp.float32),
        jax.ShapeDtypeStruct((B, S), jnp.float32),
        jax.ShapeDtypeStruct((B, S), jnp.float32),
    )
    return pl.pallas_call(
        _fps_body,
        out_shape=out_shapes,
    )(init_far, xT[:, 0], xT[:, 1], xT[:, 2])


def _px_body(pts_ref, a8_ref, w1p_ref, w1x_ref, b1_ref, out_ref):
    acc = jnp.dot(pts_ref[...], w1p_ref[...],
                  preferred_element_type=jnp.float32)
    acc = acc + jnp.dot(a8_ref[...], w1x_ref[...],
                        preferred_element_type=jnp.float32)
    bq = (acc + b1_ref[...]).astype(jnp.bfloat16).astype(jnp.float32)
    u = lax.bitcast_convert_type(bq, jnp.uint32)
    lo = u[:, :128] >> 16
    hi = (u[:, 128:] >> 16) << 16
    out_ref[...] = hi | lo


def _run_px(points_flat, xyz8, W1p, W1x8, b1):
    bm = 1024
    grid = (B * N // bm,)
    return pl.pallas_call(
        _px_body,
        grid=grid,
        compiler_params=pltpu.CompilerParams(
            dimension_semantics=("parallel",)),
        in_specs=[
            pl.BlockSpec((bm, F_IN), lambda i: (i, 0)),
            pl.BlockSpec((bm, 8), lambda i: (i, 0)),
            pl.BlockSpec((F_IN, F_MID), lambda i: (0, 0)),
            pl.BlockSpec((8, F_MID), lambda i: (0, 0)),
            pl.BlockSpec((1, F_MID), lambda i: (0, 0)),
        ],
        out_specs=pl.BlockSpec((bm, F_MID // 2), lambda i: (i, 0)),
        out_shape=jax.ShapeDtypeStruct((B * N, F_MID // 2), jnp.uint32),
    )(points_flat, xyz8, W1p, W1x8, b1)


def _bq_body(x_ref, y_ref, z_ref, xb8_ref, cx_ref, cy_ref, cz_ref, out_ref,
             s_scr):
    b = pl.program_id(0)
    x = x_ref[0]
    y = y_ref[0]
    z = z_ref[0]
    xb8 = xb8_ref[0]
    pn = (x * x + y * y) + z * z
    cb = jnp.concatenate(
        [cx_ref[0], cy_ref[0], cz_ref[0], jnp.zeros((S, 5), jnp.float32)],
        axis=1).astype(jnp.bfloat16)
    s_scr[...] = jnp.dot(cb, xb8, preferred_element_type=jnp.float32)

    RW = 32

    def chunk(sc, _):
        r = pl.ds(pl.multiple_of(sc * RW, RW), RW)
        cx = cx_ref[0, r, :]
        cy = cy_ref[0, r, :]
        cz = cz_ref[0, r, :]
        cn = (cx * cx + cy * cy) + cz * cz
        s = s_scr[r, :]
        d = (-2.0 * s + cn) + pn
        m = (d <= R2).astype(jnp.int32)
        cs = m
        sh = 1
        while sh < N:
            z128 = jnp.zeros((RW, sh), jnp.int32)
            cs = cs + jnp.concatenate([z128, cs[:, :-sh]], axis=1)
            sh *= 2
        cols = []
        for k in range(K):
            cols.append(jnp.sum((cs <= k).astype(jnp.int32), axis=1))
        cnt = jnp.stack(cols, axis=1)
        first = cnt[:, 0:1]
        sel = jnp.where(cnt == N, first, cnt) + b * N
        out_ref[0, r, :] = sel
        return 0

    lax.fori_loop(0, S // RW, chunk, 0)


def _run_bq(xT, xb8, cx, cy, cz):
    return pl.pallas_call(
        _bq_body,
        grid=(B,),
        compiler_params=pltpu.CompilerParams(
            dimension_semantics=("parallel",)),
        in_specs=[
            pl.BlockSpec((1, 1, N), lambda b: (b, 0, 0)),
            pl.BlockSpec((1, 1, N), lambda b: (b, 0, 0)),
            pl.BlockSpec((1, 1, N), lambda b: (b, 0, 0)),
            pl.BlockSpec((1, 8, N), lambda b: (b, 0, 0)),
            pl.BlockSpec((1, S, 1), lambda b: (b, 0, 0)),
            pl.BlockSpec((1, S, 1), lambda b: (b, 0, 0)),
            pl.BlockSpec((1, S, 1), lambda b: (b, 0, 0)),
        ],
        out_specs=pl.BlockSpec((1, S, K), lambda b: (b, 0, 0)),
        out_shape=jax.ShapeDtypeStruct((B, S, K), jnp.int32),
        scratch_shapes=[pltpu.VMEM((S, N), jnp.float32)],
    )(xT[:, 0:1], xT[:, 1:2], xT[:, 2:3], xb8,
      cx.reshape(B, S, 1), cy.reshape(B, S, 1), cz.reshape(B, S, 1))


def _run_sc_gather(px_flat, gidx_flat, pts_flat, fps_flat):
    NW = 32
    G_ROWS = B * S * K
    CH = 128
    g_per_w = G_ROWS // NW
    id_per_w = B * S // NW
    mesh = plsc.VectorSubcoreMesh(core_axis_name="c", subcore_axis_name="s")

    @functools.partial(
        pl.kernel,
        mesh=mesh,
        out_type=[
            jax.ShapeDtypeStruct((G_ROWS, F_MID // 2), jnp.uint32),
            jax.ShapeDtypeStruct((B * S, F_IN), jnp.float32),
        ],
        scratch_types=[
            pltpu.VMEM((CH,), jnp.int32),
            pltpu.VMEM((CH, F_MID // 2), jnp.uint32),
            pltpu.VMEM((id_per_w,), jnp.int32),
            pltpu.VMEM((id_per_w, F_IN), jnp.float32),
            pltpu.SemaphoreType.DMA,
        ],
    )
    def sc_gather(px_hbm, gidx_hbm, pts_hbm, fps_hbm, g_out, id_out,
                  idx_v, rows_v, idx2_v, rows2_v, sem):
        wid = lax.axis_index("s") * 2 + lax.axis_index("c")

        base2 = wid * id_per_w
        pltpu.sync_copy(fps_hbm.at[pl.ds(base2, id_per_w)], idx2_v)
        pltpu.async_copy(pts_hbm.at[idx2_v], rows2_v, sem).wait()
        pltpu.sync_copy(rows2_v, id_out.at[pl.ds(base2, id_per_w)])

        def body(c, _):
            base = wid * g_per_w + c * CH
            pltpu.sync_copy(gidx_hbm.at[pl.ds(base, CH)], idx_v)
            pltpu.async_copy(px_hbm.at[idx_v], rows_v, sem).wait()
            pltpu.sync_copy(rows_v, g_out.at[pl.ds(base, CH)])
            return 0

        lax.fori_loop(0, g_per_w // CH, body, 0)

    return sc_gather(px_flat, gidx_flat, pts_flat, fps_flat)


def _mlp_body(g_ref, n8_ref, id_ref, w1x_ref, w2_ref, wsk_ref, b2_ref,
              bsk_ref, out_ref):
    SB = n8_ref.shape[1]
    c1 = jnp.dot(n8_ref[0], w1x_ref[...],
                 preferred_element_type=jnp.float32)
    gu = g_ref[0]
    f_lo = lax.bitcast_convert_type(gu << 16, jnp.float32)
    f_hi = lax.bitcast_convert_type((gu >> 16) << 16, jnp.float32)
    g = jnp.concatenate([f_lo, f_hi], axis=1).reshape(SB, K, F_MID)
    h1 = jnp.maximum(g - c1[:, None, :], 0.0).reshape(SB * K, F_MID)
    h2 = jnp.dot(h1.astype(jnp.bfloat16), w2_ref[...].astype(jnp.bfloat16),
                 preferred_element_type=jnp.float32)
    m = jnp.max(h2.reshape(SB, K, F_OUT), axis=1)
    skip = jnp.dot(id_ref[0].astype(jnp.bfloat16),
                   wsk_ref[...].astype(jnp.bfloat16),
                   preferred_element_type=jnp.float32)
    out_ref[0] = jnp.maximum(m + b2_ref[...] + skip + bsk_ref[...], 0.0)


def _run_mlp(g, nxyz8, identity, W1x8, W2, Wskip, b2, bskip):
    SB = 32
    grid = (B, S // SB)
    return pl.pallas_call(
        _mlp_body,
        grid=grid,
        compiler_params=pltpu.CompilerParams(
            dimension_semantics=("parallel", "parallel")),
        in_specs=[
            pl.BlockSpec((1, SB * K, F_MID // 2), lambda b, s: (b, s, 0)),
            pl.BlockSpec((1, SB, 8), lambda b, s: (b, s, 0)),
            pl.BlockSpec((1, SB, F_IN), lambda b, s: (b, s, 0)),
            pl.BlockSpec((8, F_MID), lambda b, s: (0, 0)),
            pl.BlockSpec((F_MID, F_OUT), lambda b, s: (0, 0)),
            pl.BlockSpec((F_IN, F_OUT), lambda b, s: (0, 0)),
            pl.BlockSpec((1, F_OUT), lambda b, s: (0, 0)),
            pl.BlockSpec((1, F_OUT), lambda b, s: (0, 0)),
        ],
        out_specs=pl.BlockSpec((1, SB, F_OUT), lambda b, s: (b, s, 0)),
        out_shape=jax.ShapeDtypeStruct((B, S, F_OUT), jnp.float32),
    )(g, nxyz8, identity, W1x8, W2, Wskip, b2, bskip)


def kernel(xyz, points, W1, b1, W2, b2, Wskip, bskip):
    fps_key = jax.random.key(42)
    init_far = jax.random.randint(fps_key, (B,), 0, N, dtype=jnp.int32)
    init_far = jnp.broadcast_to(init_far[:, None], (B, 128))

    xT = jnp.transpose(xyz, (0, 2, 1))
    fps_idx, cx, cy, cz = _run_fps(xT, init_far)
    new_xyz = jnp.stack([cx, cy, cz], axis=-1)

    W1x = W1[:3]
    W1p = W1[3:]
    W1x8 = jnp.pad(W1x, ((0, 5), (0, 0)))
    xyz8 = jnp.pad(xyz.reshape(B * N, 3), ((0, 0), (0, 5)))
    px = _run_px(points.reshape(B * N, F_IN), xyz8, W1p, W1x8,
                 b1.reshape(1, F_MID))

    xb8 = jnp.concatenate(
        [xT.astype(jnp.bfloat16),
         jnp.zeros((B, 5, N), jnp.bfloat16)], axis=1)
    gidx = _run_bq(xT, xb8, cx, cy, cz)
    fps_flat = (fps_idx + jnp.arange(B, dtype=jnp.int32)[:, None] * N)

    g_flat, id_flat = _run_sc_gather(
        px, gidx.reshape(B * S * K), points.reshape(B * N, F_IN),
        fps_flat.reshape(B * S))

    nxyz8 = jnp.pad(new_xyz, ((0, 0), (0, 0), (0, 5)))
    x = _run_mlp(g_flat.reshape(B, S * K, F_MID // 2), nxyz8,
                 id_flat.reshape(B, S, F_IN), W1x8, W2, Wskip,
                 b2.reshape(1, F_OUT), bskip.reshape(1, F_OUT))
    return (new_xyz, x)


# speedup vs baseline: 1.3101x; 1.0184x over previous
import functools

import jax
import jax.numpy as jnp
from jax import lax
from jax.experimental import pallas as pl
from jax.experimental.pallas import tpu as pltpu
from jax.experimental.pallas import tpu_sc as plsc

B = 8
N = 4096
S = 512
K = 32
R2 = 0.2 ** 2
F_IN = 128
F_MID = 256
F_OUT = 512


def _fps_body(initf_ref, x_ref, y_ref, z_ref, idx_ref, cx_ref, cy_ref, cz_ref):
    x = x_ref[...]
    y = y_ref[...]
    z = z_ref[...]
    iota = lax.broadcasted_iota(jnp.int32, (B, N), 1)
    iota_s = lax.broadcasted_iota(jnp.int32, (B, S), 1)

    def step(i, carry):
        distance, far, aidx, acx, acy, acz = carry
        onehot = iota == far
        cx = jnp.sum(jnp.where(onehot, x, 0.0), axis=1, keepdims=True)
        cy = jnp.sum(jnp.where(onehot, y, 0.0), axis=1, keepdims=True)
        cz = jnp.sum(jnp.where(onehot, z, 0.0), axis=1, keepdims=True)
        hit = iota_s == i
        aidx = jnp.where(hit, far, aidx)
        acx = jnp.where(hit, cx, acx)
        acy = jnp.where(hit, cy, acy)
        acz = jnp.where(hit, cz, acz)
        dx = x - cx
        dy = y - cy
        dz = z - cz
        d = (dx * dx + dy * dy) + dz * dz
        distance = jnp.minimum(distance, d)
        m = jnp.max(distance, axis=1, keepdims=True)
        far_new = jnp.min(jnp.where(distance == m, iota, N), axis=1,
                          keepdims=True)
        return distance, far_new, aidx, acx, acy, acz

    dist0 = jnp.full((B, N), 1e10, dtype=jnp.float32)
    far0 = initf_ref[:, 0:1]
    zi = jnp.zeros((B, S), jnp.int32)
    zf = jnp.zeros((B, S), jnp.float32)
    _, _, aidx, acx, acy, acz = lax.fori_loop(
        0, S, step, (dist0, far0, zi, zf, zf, zf))
    idx_ref[...] = aidx
    cx_ref[...] = acx
    cy_ref[...] = acy
    cz_ref[...] = acz


def _run_fps(xT, init_far):
    out_shapes = (
        jax.ShapeDtypeStruct((B, S), jnp.int32),
        jax.ShapeDtypeStruct((B, S), jnp.float32),
        jax.ShapeDtypeStruct((B, S), jnp.float32),
        jax.ShapeDtypeStruct((B, S), jnp.float32),
    )
    return pl.pallas_call(
        _fps_body,
        out_shape=out_shapes,
    )(init_far, xT[:, 0], xT[:, 1], xT[:, 2])


def _px_body(pts_ref, a8_ref, w1p_ref, w1x_ref, b1_ref, out_ref):
    acc = jnp.dot(pts_ref[...], w1p_ref[...],
                  preferred_element_type=jnp.float32)
    acc = acc + jnp.dot(a8_ref[...], w1x_ref[...],
                        preferred_element_type=jnp.float32)
    bq = (acc + b1_ref[...]).astype(jnp.bfloat16).astype(jnp.float32)
    u = lax.bitcast_convert_type(bq, jnp.uint32)
    lo = u[:, :128] >> 16
    hi = (u[:, 128:] >> 16) << 16
    out_ref[...] = hi | lo


def _run_px(points_flat, xyz8, W1p, W1x8, b1):
    bm = 1024
    grid = (B * N // bm,)
    return pl.pallas_call(
        _px_body,
        grid=grid,
        compiler_params=pltpu.CompilerParams(
            dimension_semantics=("parallel",)),
        in_specs=[
            pl.BlockSpec((bm, F_IN), lambda i: (i, 0)),
            pl.BlockSpec((bm, 8), lambda i: (i, 0)),
            pl.BlockSpec((F_IN, F_MID), lambda i: (0, 0)),
            pl.BlockSpec((8, F_MID), lambda i: (0, 0)),
            pl.BlockSpec((1, F_MID), lambda i: (0, 0)),
        ],
        out_specs=pl.BlockSpec((bm, F_MID // 2), lambda i: (i, 0)),
        out_shape=jax.ShapeDtypeStruct((B * N, F_MID // 2), jnp.uint32),
    )(points_flat, xyz8, W1p, W1x8, b1)


def _bq_body(x_ref, y_ref, z_ref, xb8_ref, cx_ref, cy_ref, cz_ref, out_ref,
             s_scr):
    b = pl.program_id(0)
    x = x_ref[0]
    y = y_ref[0]
    z = z_ref[0]
    xb8 = xb8_ref[0]
    pn = (x * x + y * y) + z * z
    cb = jnp.concatenate(
        [cx_ref[0], cy_ref[0], cz_ref[0], jnp.zeros((S, 5), jnp.float32)],
        axis=1).astype(jnp.bfloat16)
    s_scr[...] = jnp.dot(cb, xb8, preferred_element_type=jnp.float32)

    RW = 64

    def chunk(sc, _):
        r = pl.ds(pl.multiple_of(sc * RW, RW), RW)
        cx = cx_ref[0, r, :]
        cy = cy_ref[0, r, :]
        cz = cz_ref[0, r, :]
        cn = (cx * cx + cy * cy) + cz * cz
        s = s_scr[r, :]
        d = (-2.0 * s + cn) + pn
        m = (d <= R2).astype(jnp.int32)
        cs = m
        sh = 1
        while sh < N:
            z128 = jnp.zeros((RW, sh), jnp.int32)
            cs = cs + jnp.concatenate([z128, cs[:, :-sh]], axis=1)
            sh *= 2
        cols = []
        for k in range(K):
            cols.append(jnp.sum((cs <= k).astype(jnp.int32), axis=1))
        cnt = jnp.stack(cols, axis=1)
        first = cnt[:, 0:1]
        sel = jnp.where(cnt == N, first, cnt) + b * N
        out_ref[0, r, :] = sel
        return 0

    lax.fori_loop(0, S // RW, chunk, 0)


def _run_bq(xT, xb8, cx, cy, cz):
    return pl.pallas_call(
        _bq_body,
        grid=(B,),
        compiler_params=pltpu.CompilerParams(
            dimension_semantics=("parallel",)),
        in_specs=[
            pl.BlockSpec((1, 1, N), lambda b: (b, 0, 0)),
            pl.BlockSpec((1, 1, N), lambda b: (b, 0, 0)),
            pl.BlockSpec((1, 1, N), lambda b: (b, 0, 0)),
            pl.BlockSpec((1, 8, N), lambda b: (b, 0, 0)),
            pl.BlockSpec((1, S, 1), lambda b: (b, 0, 0)),
            pl.BlockSpec((1, S, 1), lambda b: (b, 0, 0)),
            pl.BlockSpec((1, S, 1), lambda b: (b, 0, 0)),
        ],
        out_specs=pl.BlockSpec((1, S, K), lambda b: (b, 0, 0)),
        out_shape=jax.ShapeDtypeStruct((B, S, K), jnp.int32),
        scratch_shapes=[pltpu.VMEM((S, N), jnp.float32)],
    )(xT[:, 0:1], xT[:, 1:2], xT[:, 2:3], xb8,
      cx.reshape(B, S, 1), cy.reshape(B, S, 1), cz.reshape(B, S, 1))


def _run_sc_gather(px_flat, gidx_flat, pts_flat, fps_flat):
    NW = 32
    G_ROWS = B * S * K
    CH = 128
    g_per_w = G_ROWS // NW
    id_per_w = B * S // NW
    mesh = plsc.VectorSubcoreMesh(core_axis_name="c", subcore_axis_name="s")

    @functools.partial(
        pl.kernel,
        mesh=mesh,
        out_type=[
            jax.ShapeDtypeStruct((G_ROWS, F_MID // 2), jnp.uint32),
            jax.ShapeDtypeStruct((B * S, F_IN), jnp.float32),
        ],
        scratch_types=[
            pltpu.VMEM((CH,), jnp.int32),
            pltpu.VMEM((CH, F_MID // 2), jnp.uint32),
            pltpu.VMEM((id_per_w,), jnp.int32),
            pltpu.VMEM((id_per_w, F_IN), jnp.float32),
            pltpu.SemaphoreType.DMA,
        ],
    )
    def sc_gather(px_hbm, gidx_hbm, pts_hbm, fps_hbm, g_out, id_out,
                  idx_v, rows_v, idx2_v, rows2_v, sem):
        wid = lax.axis_index("s") * 2 + lax.axis_index("c")

        base2 = wid * id_per_w
        pltpu.sync_copy(fps_hbm.at[pl.ds(base2, id_per_w)], idx2_v)
        pltpu.async_copy(pts_hbm.at[idx2_v], rows2_v, sem).wait()
        pltpu.sync_copy(rows2_v, id_out.at[pl.ds(base2, id_per_w)])

        def body(c, _):
            base = wid * g_per_w + c * CH
            pltpu.sync_copy(gidx_hbm.at[pl.ds(base, CH)], idx_v)
            pltpu.async_copy(px_hbm.at[idx_v], rows_v, sem).wait()
            pltpu.sync_copy(rows_v, g_out.at[pl.ds(base, CH)])
            return 0

        lax.fori_loop(0, g_per_w // CH, body, 0)

    return sc_gather(px_flat, gidx_flat, pts_flat, fps_flat)


def _mlp_body(g_ref, n8_ref, id_ref, w1x_ref, w2_ref, wsk_ref, b2_ref,
              bsk_ref, out_ref):
    SB = n8_ref.shape[1]
    c1 = jnp.dot(n8_ref[0], w1x_ref[...],
                 preferred_element_type=jnp.float32)
    gu = g_ref[0]
    f_lo = lax.bitcast_convert_type(gu << 16, jnp.float32)
    f_hi = lax.bitcast_convert_type((gu >> 16) << 16, jnp.float32)
    g = jnp.concatenate([f_lo, f_hi], axis=1).reshape(SB, K, F_MID)
    h1 = jnp.maximum(g - c1[:, None, :], 0.0).reshape(SB * K, F_MID)
    h2 = jnp.dot(h1.astype(jnp.bfloat16), w2_ref[...].astype(jnp.bfloat16),
                 preferred_element_type=jnp.float32)
    m = jnp.max(h2.reshape(SB, K, F_OUT), axis=1)
    skip = jnp.dot(id_ref[0].astype(jnp.bfloat16),
                   wsk_ref[...].astype(jnp.bfloat16),
                   preferred_element_type=jnp.float32)
    out_ref[0] = jnp.maximum(m + b2_ref[...] + skip + bsk_ref[...], 0.0)


def _run_mlp(g, nxyz8, identity, W1x8, W2, Wskip, b2, bskip):
    SB = 32
    grid = (B, S // SB)
    return pl.pallas_call(
        _mlp_body,
        grid=grid,
        compiler_params=pltpu.CompilerParams(
            dimension_semantics=("parallel", "parallel")),
        in_specs=[
            pl.BlockSpec((1, SB * K, F_MID // 2), lambda b, s: (b, s, 0)),
            pl.BlockSpec((1, SB, 8), lambda b, s: (b, s, 0)),
            pl.BlockSpec((1, SB, F_IN), lambda b, s: (b, s, 0)),
            pl.BlockSpec((8, F_MID), lambda b, s: (0, 0)),
            pl.BlockSpec((F_MID, F_OUT), lambda b, s: (0, 0)),
            pl.BlockSpec((F_IN, F_OUT), lambda b, s: (0, 0)),
            pl.BlockSpec((1, F_OUT), lambda b, s: (0, 0)),
            pl.BlockSpec((1, F_OUT), lambda b, s: (0, 0)),
        ],
        out_specs=pl.BlockSpec((1, SB, F_OUT), lambda b, s: (b, s, 0)),
        out_shape=jax.ShapeDtypeStruct((B, S, F_OUT), jnp.float32),
    )(g, nxyz8, identity, W1x8, W2, Wskip, b2, bskip)


def kernel(xyz, points, W1, b1, W2, b2, Wskip, bskip):
    fps_key = jax.random.key(42)
    init_far = jax.random.randint(fps_key, (B,), 0, N, dtype=jnp.int32)
    init_far = jnp.broadcast_to(init_far[:, None], (B, 128))

    xT = jnp.transpose(xyz, (0, 2, 1))
    fps_idx, cx, cy, cz = _run_fps(xT, init_far)
    new_xyz = jnp.stack([cx, cy, cz], axis=-1)

    W1x = W1[:3]
    W1p = W1[3:]
    W1x8 = jnp.pad(W1x, ((0, 5), (0, 0)))
    xyz8 = jnp.pad(xyz.reshape(B * N, 3), ((0, 0), (0, 5)))
    px = _run_px(points.reshape(B * N, F_IN), xyz8, W1p, W1x8,
                 b1.reshape(1, F_MID))

    xb8 = jnp.concatenate(
        [xT.astype(jnp.bfloat16),
         jnp.zeros((B, 5, N), jnp.bfloat16)], axis=1)
    gidx = _run_bq(xT, xb8, cx, cy, cz)
    fps_flat = (fps_idx + jnp.arange(B, dtype=jnp.int32)[:, None] * N)

    g_flat, id_flat = _run_sc_gather(
        px, gidx.reshape(B * S * K), points.reshape(B * N, F_IN),
        fps_flat.reshape(B * S))

    nxyz8 = jnp.pad(new_xyz, ((0, 0), (0, 0), (0, 5)))
    x = _run_mlp(g_flat.reshape(B, S * K, F_MID // 2), nxyz8,
                 id_flat.reshape(B, S, F_IN), W1x8, W2, Wskip,
                 b2.reshape(1, F_OUT), bskip.reshape(1, F_OUT))
    return (new_xyz, x)
